# baseline (speedup 1.0000x reference)
import jax
import jax.numpy as jnp
from jax.experimental import pallas as pl
from jax.experimental.pallas import tpu as pltpu

_BATCH_SIZE = 256
_ROWS_PER_BLOCK = 1000
_CHUNK = 200


def _thresh_body(qT_ref, tT_ref, c_ref):
    qT = qT_ref[...]
    tT = tT_ref[...]
    qt = jnp.sum(qT * tT, axis=0, keepdims=True)
    tn = jnp.sqrt(jnp.sum(tT * tT, axis=0, keepdims=True))
    c_ref[...] = qt / jnp.maximum(tn, 1e-12)


def _rank_body(qidx_ref, data_ref, qT_ref, c_ref, out_ref):
    i = pl.program_id(0)
    q = out_ref.shape[1]

    @pl.when(i == 0)
    def _init():
        out_ref[...] = jnp.zeros_like(out_ref)

    qbase = qidx_ref[0] * _BATCH_SIZE
    for j in range(_ROWS_PER_BLOCK // _CHUNK):
        d = data_ref[j * _CHUNK:(j + 1) * _CHUNK, :]
        raw = jnp.dot(d.astype(jnp.bfloat16), qT_ref[...],
                      preferred_element_type=jnp.float32)
        dn = jnp.sqrt(jnp.sum(d * d, axis=1, keepdims=True))
        dn = jnp.maximum(dn, 1e-12)
        hit = raw >= c_ref[...] * dn
        out_ref[...] += jnp.sum(hit.astype(jnp.int32), axis=0, keepdims=True)

        chunk_lo = i * _ROWS_PER_BLOCK + j * _CHUNK

        @pl.when((qbase < chunk_lo + _CHUNK) & (qbase + q > chunk_lo))
        def _self_mask(raw=raw, hit=hit, chunk_lo=chunk_lo):
            row_ids = chunk_lo + jax.lax.broadcasted_iota(
                jnp.int32, raw.shape, 0)
            q_ids = qbase + jax.lax.broadcasted_iota(jnp.int32, raw.shape, 1)
            corr = hit & (row_ids == q_ids)
            out_ref[...] -= jnp.sum(corr.astype(jnp.int32), axis=0,
                                    keepdims=True)


def kernel(data, queries, truths, query_idx_in_rankingset):
    n, k = data.shape
    q = queries.shape[0]
    r = _ROWS_PER_BLOCK
    assert n % r == 0
    qTf = queries.T
    tTf = truths.T
    c = pl.pallas_call(
        _thresh_body,
        out_shape=jax.ShapeDtypeStruct((1, q), jnp.float32),
    )(qTf, tTf)
    qT = qTf.astype(jnp.bfloat16)
    qidx = jnp.asarray(query_idx_in_rankingset, jnp.int32).reshape(1)
    return pl.pallas_call(
        _rank_body,
        grid_spec=pltpu.PrefetchScalarGridSpec(
            num_scalar_prefetch=1,
            grid=(n // r,),
            in_specs=[
                pl.BlockSpec((r, k), lambda i, s: (i, 0)),
                pl.BlockSpec((k, q), lambda i, s: (0, 0)),
                pl.BlockSpec((1, q), lambda i, s: (0, 0)),
            ],
            out_specs=pl.BlockSpec((1, q), lambda i, s: (0, 0)),
        ),
        out_shape=jax.ShapeDtypeStruct((1, q), jnp.int32),
        compiler_params=pltpu.CompilerParams(
            dimension_semantics=("arbitrary",)),
    )(qidx, data, qT, c)


# speedup vs baseline: 1.0603x; 1.0603x over previous
import jax
import jax.numpy as jnp
from jax.experimental import pallas as pl
from jax.experimental.pallas import tpu as pltpu

_BATCH_SIZE = 256
_ROWS_PER_BLOCK = 400
_CHUNK = 400


def _thresh_body(qT_ref, tT_ref, c_ref):
    qT = qT_ref[...]
    tT = tT_ref[...]
    qt = jnp.sum(qT * tT, axis=0, keepdims=True)
    tn = jnp.sqrt(jnp.sum(tT * tT, axis=0, keepdims=True))
    c_ref[...] = qt / jnp.maximum(tn, 1e-12)


def _rank_body(qidx_ref, data_ref, qT_ref, c_ref, out_ref):
    i = pl.program_id(0)
    q = out_ref.shape[1]

    @pl.when(i == 0)
    def _init():
        out_ref[...] = jnp.zeros_like(out_ref)

    qbase = qidx_ref[0] * _BATCH_SIZE
    for j in range(_ROWS_PER_BLOCK // _CHUNK):
        d = data_ref[j * _CHUNK:(j + 1) * _CHUNK, :]
        raw = jnp.dot(d.astype(jnp.bfloat16), qT_ref[...],
                      preferred_element_type=jnp.float32)
        dn = jnp.sqrt(jnp.sum(d * d, axis=1, keepdims=True))
        dn = jnp.maximum(dn, 1e-12)
        hit = raw >= c_ref[...] * dn
        out_ref[...] += jnp.sum(hit.astype(jnp.int32), axis=0, keepdims=True)

        chunk_lo = i * _ROWS_PER_BLOCK + j * _CHUNK

        @pl.when((qbase < chunk_lo + _CHUNK) & (qbase + q > chunk_lo))
        def _self_mask(raw=raw, hit=hit, chunk_lo=chunk_lo):
            row_ids = chunk_lo + jax.lax.broadcasted_iota(
                jnp.int32, raw.shape, 0)
            q_ids = qbase + jax.lax.broadcasted_iota(jnp.int32, raw.shape, 1)
            corr = hit & (row_ids == q_ids)
            out_ref[...] -= jnp.sum(corr.astype(jnp.int32), axis=0,
                                    keepdims=True)


def kernel(data, queries, truths, query_idx_in_rankingset):
    n, k = data.shape
    q = queries.shape[0]
    r = _ROWS_PER_BLOCK
    assert n % r == 0
    qTf = queries.T
    tTf = truths.T
    c = pl.pallas_call(
        _thresh_body,
        out_shape=jax.ShapeDtypeStruct((1, q), jnp.float32),
    )(qTf, tTf)
    qT = qTf.astype(jnp.bfloat16)
    qidx = jnp.asarray(query_idx_in_rankingset, jnp.int32).reshape(1)
    return pl.pallas_call(
        _rank_body,
        grid_spec=pltpu.PrefetchScalarGridSpec(
            num_scalar_prefetch=1,
            grid=(n // r,),
            in_specs=[
                pl.BlockSpec((r, k), lambda i, s: (i, 0)),
                pl.BlockSpec((k, q), lambda i, s: (0, 0)),
                pl.BlockSpec((1, q), lambda i, s: (0, 0)),
            ],
            out_specs=pl.BlockSpec((1, q), lambda i, s: (0, 0)),
        ),
        out_shape=jax.ShapeDtypeStruct((1, q), jnp.int32),
        compiler_params=pltpu.CompilerParams(
            dimension_semantics=("arbitrary",)),
    )(qidx, data, qT, c)


# speedup vs baseline: 1.1768x; 1.1099x over previous
import jax
import jax.numpy as jnp
from jax.experimental import pallas as pl
from jax.experimental.pallas import tpu as pltpu

_BATCH_SIZE = 256
_ROWS_PER_BLOCK = 400


def _rank_body(qidx_ref, data_ref, qT_ref, tT_ref, out_ref, c_ref):
    i = pl.program_id(0)
    r, q = _ROWS_PER_BLOCK, out_ref.shape[1]

    @pl.when(i == 0)
    def _init():
        qT = qT_ref[...].astype(jnp.float32)
        tT = tT_ref[...].astype(jnp.float32)
        qt = jnp.sum(qT * tT, axis=0, keepdims=True)
        tn = jnp.sqrt(jnp.sum(tT * tT, axis=0, keepdims=True))
        c_ref[...] = qt / jnp.maximum(tn, 1e-12)
        out_ref[...] = jnp.zeros_like(out_ref)

    d_bf = data_ref[...].astype(jnp.bfloat16)
    raw = jnp.dot(d_bf, qT_ref[...],
                  preferred_element_type=jnp.float32)
    df = d_bf.astype(jnp.float32)
    dn = jnp.sqrt(jnp.sum(df * df, axis=1, keepdims=True))
    dn = jnp.maximum(dn, 1e-12)
    hit = raw >= c_ref[...] * dn
    out_ref[...] += jnp.sum(hit.astype(jnp.int32), axis=0, keepdims=True)

    qbase = qidx_ref[0] * _BATCH_SIZE
    blk_lo = i * r

    @pl.when((qbase < blk_lo + r) & (qbase + q > blk_lo))
    def _self_mask():
        row_ids = blk_lo + jax.lax.broadcasted_iota(jnp.int32, raw.shape, 0)
        q_ids = qbase + jax.lax.broadcasted_iota(jnp.int32, raw.shape, 1)
        corr = hit & (row_ids == q_ids)
        out_ref[...] -= jnp.sum(corr.astype(jnp.int32), axis=0, keepdims=True)


def kernel(data, queries, truths, query_idx_in_rankingset):
    n, k = data.shape
    q = queries.shape[0]
    r = _ROWS_PER_BLOCK
    assert n % r == 0
    qT = queries.T.astype(jnp.bfloat16)
    tT = truths.T.astype(jnp.bfloat16)
    qidx = jnp.asarray(query_idx_in_rankingset, jnp.int32).reshape(1)
    return pl.pallas_call(
        _rank_body,
        grid_spec=pltpu.PrefetchScalarGridSpec(
            num_scalar_prefetch=1,
            grid=(n // r,),
            in_specs=[
                pl.BlockSpec((r, k), lambda i, s: (i, 0)),
                pl.BlockSpec((k, q), lambda i, s: (0, 0)),
                pl.BlockSpec((k, q), lambda i, s: (0, 0)),
            ],
            out_specs=pl.BlockSpec((1, q), lambda i, s: (0, 0)),
            scratch_shapes=[pltpu.VMEM((1, q), jnp.float32)],
        ),
        out_shape=jax.ShapeDtypeStruct((1, q), jnp.int32),
        compiler_params=pltpu.CompilerParams(
            dimension_semantics=("arbitrary",)),
    )(qidx, data, qT, tT)
